# Initial kernel scaffold; baseline (speedup 1.0000x reference)
import jax
import jax.numpy as jnp
from jax.experimental import pallas as pl


def kernel(logits, img_feats, text_norm_feats, gate_mask):
    raise NotImplementedError("write your pallas kernel here")


# speedup vs baseline: 1.2789x; 1.2789x over previous
import functools

import jax
import jax.numpy as jnp
from jax import lax
from jax.experimental import pallas as pl
from jax.experimental.pallas import tpu as pltpu
from jax.experimental.pallas import tpu_sc as plsc


def _labels_body(C, logits_ref, gate_ref, out_ref):
    x = logits_ref[...]
    m = jnp.max(x, axis=1, keepdims=True)
    col = lax.broadcasted_iota(jnp.int32, x.shape, 1)
    idx = jnp.min(jnp.where(x == m, col, C), axis=1)
    g = gate_ref[0, 0, :]
    out_ref[0, 0, :] = jnp.where(g > 0, idx, C).reshape(1, 1, -1)[0, 0, :]


def _compute_labels(logits, gate_i32, blk):
    N, C = logits.shape
    nb = N // blk
    gate3 = gate_i32.reshape(nb, 1, blk)
    return pl.pallas_call(
        functools.partial(_labels_body, C),
        grid=(nb,),
        in_specs=[
            pl.BlockSpec((blk, C), lambda i: (i, 0)),
            pl.BlockSpec((1, 1, blk), lambda i: (i, 0, 0)),
        ],
        out_specs=pl.BlockSpec((1, 1, blk), lambda i: (i, 0, 0)),
        out_shape=jax.ShapeDtypeStruct((nb, 1, blk), jnp.int32),
    )(logits, gate3)


def _make_segment_sum(N, D, CP, chunk):
    info = plsc.get_sparse_core_info()
    nc, ns = info.num_cores, info.num_subcores
    rows_per_tile = N // (nc * ns)
    n_chunks = rows_per_tile // chunk
    lrows = chunk // 128
    tile_lrows = rows_per_tile // 128

    mesh = plsc.VectorSubcoreMesh(core_axis_name="c", subcore_axis_name="s")

    @functools.partial(
        pl.kernel,
        mesh=mesh,
        out_type=[
            jax.ShapeDtypeStruct((nc, CP, D), jnp.float32),
            jax.ShapeDtypeStruct((nc, CP, D), jnp.float32),
        ],
        scratch_types=[
            pltpu.VMEM((tile_lrows, 128), jnp.int32),
            pltpu.VMEM((chunk, D), jnp.float32),
            pltpu.VMEM((chunk, D), jnp.float32),
            pltpu.VMEM_SHARED((CP, D), jnp.float32),
            pltpu.VMEM_SHARED((CP, D), jnp.float32),
        ],
    )
    def seg(lbl_hbm, img_hbm, zsum_hbm, zcnt_hbm, ones_hbm,
            sums_out, cnts_out, lbl_v, img_v, ones_v, sums_sh, cnts_sh):
        cid = lax.axis_index("c")
        sid = lax.axis_index("s")

        @pl.when(sid == 0)
        def _():
            pltpu.sync_copy(zsum_hbm, sums_sh)
            pltpu.sync_copy(zcnt_hbm, cnts_sh)

        pltpu.sync_copy(ones_hbm, ones_v)
        plsc.subcore_barrier()

        rbase = (cid * ns + sid) * rows_per_tile
        lb = pl.multiple_of(rbase // 128, tile_lrows)
        pltpu.sync_copy(lbl_hbm.at[pl.ds(lb, tile_lrows)], lbl_v)
        for j in range(n_chunks):
            r0 = pl.multiple_of(rbase + j * chunk, chunk)
            pltpu.sync_copy(img_hbm.at[pl.ds(r0, chunk)], img_v)
            for k in range(lrows):
                idx = lbl_v.at[j * lrows + k]
                src = img_v.at[pl.ds(k * 128, 128)]
                pltpu.sync_copy(src, sums_sh.at[idx], add=True)
                pltpu.sync_copy(ones_v.at[pl.ds(k * 128, 128)],
                                cnts_sh.at[idx], add=True)

        plsc.subcore_barrier()

        @pl.when(sid == 0)
        def _():
            pltpu.sync_copy(sums_sh, sums_out.at[cid])
            pltpu.sync_copy(cnts_sh, cnts_out.at[cid])

    return seg


def _final_body(C, sums_ref, cnts_ref, text_ref, out_ref):
    s = sums_ref[0] + sums_ref[1]
    cnt = cnts_ref[0, :, 0] + cnts_ref[1, :, 0]
    CP = s.shape[0]
    rows = lax.broadcasted_iota(jnp.int32, (CP,), 0)
    valid = (cnt > 0.0) & (rows < C)
    safe = jnp.where(cnt > 0.0, cnt, 1.0)
    means = s / safe[:, None]
    d = jnp.sum(means * text_ref[...], axis=1)
    num_present = jnp.sum(valid.astype(jnp.float32))
    loss = jnp.sum(jnp.where(valid, d, 0.0)) / jnp.maximum(num_present, 1.0)
    out_ref[...] = loss.reshape(1, 1)


def _finalize(sums2, cnts2, text_pad, C):
    return pl.pallas_call(
        functools.partial(_final_body, C),
        out_shape=jax.ShapeDtypeStruct((1, 1), jnp.float32),
    )(sums2, cnts2, text_pad)


@jax.jit
def kernel(logits, img_feats, text_norm_feats, gate_mask):
    N, C = logits.shape
    D = img_feats.shape[1]
    CP = ((C + 1 + 15) // 16) * 16
    BLK = 512
    CHUNK = 256

    labels3 = _compute_labels(logits, gate_mask.astype(jnp.int32), BLK)
    lbl2 = labels3.reshape(N // 128, 128)

    seg = _make_segment_sum(N, D, CP, CHUNK)
    zsum = jnp.zeros((CP, D), jnp.float32)
    zcnt = jnp.zeros((CP, D), jnp.float32)
    ones = jnp.ones((CHUNK, D), jnp.float32)
    sums2, cnts2 = seg(lbl2, img_feats, zsum, zcnt, ones)

    text_pad = jnp.pad(text_norm_feats, ((0, CP - C), (0, 0)))
    loss = _finalize(sums2, cnts2, text_pad, C)
    return loss[0, 0]
